# baseline (speedup 1.0000x reference)
import functools

import jax
import jax.numpy as jnp
from jax import lax
from jax.experimental import pallas as pl
from jax.experimental.pallas import tpu as pltpu


def _fused_mlp_kernel(x0_ref, x1_ref, x2_ref, x3_ref, w1_ref, g_ref,
                      beta_ref, w2_hbm_ref, b2_ref, o_ref, hn_ref, w2_ref,
                      w2_sem, *, eps, inv_b, n_h, t_h, t_n, nq):
    j = pl.program_id(0)

    @pl.when(j == 0)
    def _start_w2_copy():
        pltpu.make_async_copy(w2_hbm_ref, w2_ref, w2_sem).start()

    @pl.when(j < n_h)
    def _hidden_tile():
        xqs = (x0_ref, x1_ref, x2_ref, x3_ref)[:nq]
        hs = [jnp.dot(xq[...], w1_ref[...],
                      preferred_element_type=jnp.float32) for xq in xqs]
        s1 = sum(jnp.sum(h, axis=0, keepdims=True) for h in hs)
        s2 = sum(jnp.sum(h * h, axis=0, keepdims=True) for h in hs)
        mean = s1 * inv_b
        var = s2 * inv_b - mean * mean
        a = g_ref[...] * lax.rsqrt(jnp.maximum(var, 0.0) + eps)
        c = beta_ref[...] - mean * a
        col = pl.multiple_of(j * t_h, t_h)
        q = x0_ref.shape[0]
        for i, h in enumerate(hs):
            hn_ref[i * q:(i + 1) * q, pl.ds(col, t_h)] = jnp.maximum(
                h * a + c, 0.0)

    @pl.when(j == n_h)
    def _finish_w2_copy():
        pltpu.make_async_copy(w2_hbm_ref, w2_ref, w2_sem).wait()

    @pl.when(j >= n_h)
    def _out_tile():
        col = pl.multiple_of((j - n_h) * t_n, t_n)
        o_ref[...] = (jnp.dot(hn_ref[...], w2_ref[:, pl.ds(col, t_n)],
                              preferred_element_type=jnp.float32)
                      + b2_ref[...])


def kernel(w1, b1, gamma, beta, w2, b2, x):
    del b1
    B, in_dim = x.shape
    hidden = w1.shape[1]
    out_dim = w2.shape[1]
    eps = 1e-5

    g2 = gamma.reshape(1, hidden)
    beta2 = beta.reshape(1, hidden)
    b2_2 = b2.reshape(1, out_dim)

    t_h = 512 if hidden % 512 == 0 else hidden
    n_h = hidden // t_h
    t_n = 512 if out_dim % 512 == 0 else out_dim
    n_n = out_dim // t_n
    steps = n_h + n_n
    nq = 2 if B % 16 == 0 else 1
    qb = B // nq

    def w1_idx(j):
        return (0, jnp.minimum(j, n_h - 1))

    def out_idx(j):
        return (0, jnp.clip(j - n_h, 0, n_n - 1))

    def xq_idx(i):
        return lambda j: (i, 0)

    body = functools.partial(_fused_mlp_kernel, eps=eps, inv_b=1.0 / B,
                             n_h=n_h, t_h=t_h, t_n=t_n, nq=nq)
    return pl.pallas_call(
        body,
        grid=(steps,),
        in_specs=[
            pl.BlockSpec((qb, in_dim), xq_idx(0)),
            pl.BlockSpec((qb, in_dim), xq_idx(min(1, nq - 1))),
            pl.BlockSpec((qb, in_dim), xq_idx(min(2, nq - 1))),
            pl.BlockSpec((qb, in_dim), xq_idx(min(3, nq - 1))),
            pl.BlockSpec((in_dim, t_h), w1_idx),
            pl.BlockSpec((1, t_h), w1_idx),
            pl.BlockSpec((1, t_h), w1_idx),
            pl.BlockSpec(memory_space=pl.ANY),
            pl.BlockSpec((1, t_n), out_idx),
        ],
        out_specs=pl.BlockSpec((B, t_n), out_idx),
        out_shape=jax.ShapeDtypeStruct((B, out_dim), jnp.float32),
        scratch_shapes=[
            pltpu.VMEM((B, hidden), jnp.float32),
            pltpu.VMEM((hidden, out_dim), jnp.float32),
            pltpu.SemaphoreType.DMA,
        ],
        compiler_params=pltpu.CompilerParams(
            dimension_semantics=("arbitrary",)),
        cost_estimate=pl.CostEstimate(
            flops=2 * B * in_dim * hidden + 2 * B * hidden * out_dim,
            transcendentals=hidden,
            bytes_accessed=(B * in_dim + in_dim * hidden
                            + hidden * out_dim + B * out_dim) * 4,
        ),
    )(x, x, x, x, w1, g2, beta2, w2, b2_2)


# speedup vs baseline: 1.0867x; 1.0867x over previous
import functools

import jax
import jax.numpy as jnp
from jax import lax
from jax.experimental import pallas as pl
from jax.experimental.pallas import tpu as pltpu


def _fused_mlp_kernel(x0_ref, x1_ref, x2_ref, x3_ref, x4_ref, x5_ref,
                      x6_ref, x7_ref, w1_ref, g_ref,
                      beta_ref, w2_hbm_ref, b2_ref, o_ref, hn_ref, w2_ref,
                      w2_sem, *, eps, inv_b, n_h, t_h, t_n, nq):
    j = pl.program_id(0)

    @pl.when(j == 0)
    def _start_w2_copy():
        pltpu.make_async_copy(w2_hbm_ref, w2_ref, w2_sem).start()

    @pl.when(j < n_h)
    def _hidden_tile():
        xqs = (x0_ref, x1_ref, x2_ref, x3_ref, x4_ref, x5_ref, x6_ref,
               x7_ref)[:nq]
        hs = [jnp.dot(xq[...], w1_ref[...],
                      preferred_element_type=jnp.float32) for xq in xqs]
        s1 = sum(jnp.sum(h, axis=0, keepdims=True) for h in hs)
        s2 = sum(jnp.sum(h * h, axis=0, keepdims=True) for h in hs)
        mean = s1 * inv_b
        var = s2 * inv_b - mean * mean
        a = g_ref[...] * lax.rsqrt(jnp.maximum(var, 0.0) + eps)
        c = beta_ref[...] - mean * a
        col = pl.multiple_of(j * t_h, t_h)
        q = x0_ref.shape[0]
        for i, h in enumerate(hs):
            hn_ref[i * q:(i + 1) * q, pl.ds(col, t_h)] = jnp.maximum(
                h * a + c, 0.0)

    @pl.when(j == n_h)
    def _finish_w2_copy():
        pltpu.make_async_copy(w2_hbm_ref, w2_ref, w2_sem).wait()

    @pl.when(j >= n_h)
    def _out_tile():
        col = pl.multiple_of((j - n_h) * t_n, t_n)
        o_ref[...] = (jnp.dot(hn_ref[...], w2_ref[:, pl.ds(col, t_n)],
                              preferred_element_type=jnp.float32)
                      + b2_ref[...])


def kernel(w1, b1, gamma, beta, w2, b2, x):
    del b1
    B, in_dim = x.shape
    hidden = w1.shape[1]
    out_dim = w2.shape[1]
    eps = 1e-5

    g2 = gamma.reshape(1, hidden)
    beta2 = beta.reshape(1, hidden)
    b2_2 = b2.reshape(1, out_dim)

    t_h = 512 if hidden % 512 == 0 else hidden
    n_h = hidden // t_h
    t_n = 512 if out_dim % 512 == 0 else out_dim
    n_n = out_dim // t_n
    steps = n_h + n_n
    nq = 8 if B % 64 == 0 else 1
    qb = B // nq

    def w1_idx(j):
        return (0, jnp.minimum(j, n_h - 1))

    def out_idx(j):
        return (0, jnp.clip(j - n_h, 0, n_n - 1))

    def xq_idx(i):
        return lambda j: (i, 0)

    body = functools.partial(_fused_mlp_kernel, eps=eps, inv_b=1.0 / B,
                             n_h=n_h, t_h=t_h, t_n=t_n, nq=nq)
    return pl.pallas_call(
        body,
        grid=(steps,),
        in_specs=[
            *[pl.BlockSpec((qb, in_dim), xq_idx(min(i, nq - 1)))
              for i in range(8)],
            pl.BlockSpec((in_dim, t_h), w1_idx),
            pl.BlockSpec((1, t_h), w1_idx),
            pl.BlockSpec((1, t_h), w1_idx),
            pl.BlockSpec(memory_space=pl.ANY),
            pl.BlockSpec((1, t_n), out_idx),
        ],
        out_specs=pl.BlockSpec((B, t_n), out_idx),
        out_shape=jax.ShapeDtypeStruct((B, out_dim), jnp.float32),
        scratch_shapes=[
            pltpu.VMEM((B, hidden), jnp.float32),
            pltpu.VMEM((hidden, out_dim), jnp.float32),
            pltpu.SemaphoreType.DMA,
        ],
        compiler_params=pltpu.CompilerParams(
            dimension_semantics=("arbitrary",)),
        cost_estimate=pl.CostEstimate(
            flops=2 * B * in_dim * hidden + 2 * B * hidden * out_dim,
            transcendentals=hidden,
            bytes_accessed=(B * in_dim + in_dim * hidden
                            + hidden * out_dim + B * out_dim) * 4,
        ),
    )(x, x, x, x, x, x, x, x, w1, g2, beta2, w2, b2_2)


# speedup vs baseline: 1.0914x; 1.0044x over previous
import functools

import jax
import jax.numpy as jnp
from jax import lax
from jax.experimental import pallas as pl
from jax.experimental.pallas import tpu as pltpu


def _fused_mlp_kernel(x0_ref, x1_ref, x2_ref, x3_ref, w1_ref, g_ref,
                      beta_ref, w2_hbm_ref, b2_ref, o_ref, hn_ref, w2_ref,
                      w2_sem, *, eps, inv_b, n_h, t_h, t_n, nq):
    j = pl.program_id(0)

    @pl.when(j == 0)
    def _start_w2_copy():
        pltpu.make_async_copy(w2_hbm_ref, w2_ref, w2_sem).start()

    @pl.when(j < n_h)
    def _hidden_tile():
        xqs = (x0_ref, x1_ref, x2_ref, x3_ref)[:nq]
        hs = [jnp.dot(xq[...], w1_ref[...],
                      preferred_element_type=jnp.float32) for xq in xqs]
        s1 = sum(jnp.sum(h, axis=0, keepdims=True) for h in hs)
        s2 = sum(jnp.sum(h * h, axis=0, keepdims=True) for h in hs)
        mean = s1 * inv_b
        var = s2 * inv_b - mean * mean
        a = g_ref[...] * lax.rsqrt(jnp.maximum(var, 0.0) + eps)
        c = beta_ref[...] - mean * a
        col = pl.multiple_of(j * t_h, t_h)
        q = x0_ref.shape[0]
        for i, h in enumerate(hs):
            hn_ref[i * q:(i + 1) * q, pl.ds(col, t_h)] = jnp.maximum(
                h * a + c, 0.0)

    @pl.when(j == n_h)
    def _finish_w2_copy():
        pltpu.make_async_copy(w2_hbm_ref, w2_ref, w2_sem).wait()

    @pl.when(j >= n_h)
    def _out_tile():
        col = pl.multiple_of((j - n_h) * t_n, t_n)
        o_ref[...] = (jnp.dot(hn_ref[...], w2_ref[:, pl.ds(col, t_n)],
                              preferred_element_type=jnp.float32)
                      + b2_ref[...])


def kernel(w1, b1, gamma, beta, w2, b2, x):
    del b1
    B, in_dim = x.shape
    hidden = w1.shape[1]
    out_dim = w2.shape[1]
    eps = 1e-5

    g2 = gamma.reshape(1, hidden)
    beta2 = beta.reshape(1, hidden)
    b2_2 = b2.reshape(1, out_dim)

    t_h = 512 if hidden % 512 == 0 else hidden
    n_h = hidden // t_h
    t_n = 512 if out_dim % 512 == 0 else out_dim
    n_n = out_dim // t_n
    steps = n_h + n_n
    nq = 4 if B % 32 == 0 else 1
    qb = B // nq

    def w1_idx(j):
        return (0, jnp.minimum(j, n_h - 1))

    def out_idx(j):
        return (0, jnp.clip(j - n_h, 0, n_n - 1))

    def xq_idx(i):
        return lambda j: (i, 0)

    body = functools.partial(_fused_mlp_kernel, eps=eps, inv_b=1.0 / B,
                             n_h=n_h, t_h=t_h, t_n=t_n, nq=nq)
    return pl.pallas_call(
        body,
        grid=(steps,),
        in_specs=[
            *[pl.BlockSpec((qb, in_dim), xq_idx(min(i, nq - 1)))
              for i in range(4)],
            pl.BlockSpec((in_dim, t_h), w1_idx),
            pl.BlockSpec((1, t_h), w1_idx),
            pl.BlockSpec((1, t_h), w1_idx),
            pl.BlockSpec(memory_space=pl.ANY),
            pl.BlockSpec((1, t_n), out_idx),
        ],
        out_specs=pl.BlockSpec((B, t_n), out_idx),
        out_shape=jax.ShapeDtypeStruct((B, out_dim), jnp.float32),
        scratch_shapes=[
            pltpu.VMEM((B, hidden), jnp.float32),
            pltpu.VMEM((hidden, out_dim), jnp.float32),
            pltpu.SemaphoreType.DMA,
        ],
        compiler_params=pltpu.CompilerParams(
            dimension_semantics=("arbitrary",)),
        cost_estimate=pl.CostEstimate(
            flops=2 * B * in_dim * hidden + 2 * B * hidden * out_dim,
            transcendentals=hidden,
            bytes_accessed=(B * in_dim + in_dim * hidden
                            + hidden * out_dim + B * out_dim) * 4,
        ),
    )(x, x, x, x, w1, g2, beta2, w2, b2_2)


# speedup vs baseline: 1.1121x; 1.0190x over previous
import functools

import jax
import jax.numpy as jnp
from jax import lax
from jax.experimental import pallas as pl
from jax.experimental.pallas import tpu as pltpu


def _fused_mlp_kernel(x0_ref, x1_ref, x2_ref, x3_ref, w1_ref, g_ref,
                      beta_ref, w2_hbm_ref, b2_ref, o_ref, hn_ref, w2_ref,
                      w2_sem, *, eps, inv_b, n_h, t_h, t_n, nq):
    j = pl.program_id(0)

    @pl.when(j == 0)
    def _start_w2_copy():
        pltpu.make_async_copy(w2_hbm_ref, w2_ref, w2_sem).start()

    @pl.when(j < n_h)
    def _hidden_tile():
        xqs = (x0_ref, x1_ref, x2_ref, x3_ref)[:nq]
        hs = [jnp.dot(xq[...], w1_ref[...],
                      preferred_element_type=jnp.float32) for xq in xqs]
        s1 = sum(jnp.sum(h, axis=0, keepdims=True) for h in hs)
        s2 = sum(jnp.sum(h * h, axis=0, keepdims=True) for h in hs)
        mean = s1 * inv_b
        var = s2 * inv_b - mean * mean
        col = pl.multiple_of(j * t_h, t_h)
        a = g_ref[:, pl.ds(col, t_h)] * lax.rsqrt(
            jnp.maximum(var, 0.0) + eps)
        c = beta_ref[:, pl.ds(col, t_h)] - mean * a
        q = x0_ref.shape[0]
        for i, h in enumerate(hs):
            hn_ref[i * q:(i + 1) * q, pl.ds(col, t_h)] = jnp.maximum(
                h * a + c, 0.0)

    @pl.when(j == n_h)
    def _finish_w2_copy():
        pltpu.make_async_copy(w2_hbm_ref, w2_ref, w2_sem).wait()

    @pl.when(j >= n_h)
    def _out_tile():
        col = pl.multiple_of((j - n_h) * t_n, t_n)
        o_ref[...] = (jnp.dot(hn_ref[...], w2_ref[:, pl.ds(col, t_n)],
                              preferred_element_type=jnp.float32)
                      + b2_ref[:, pl.ds(col, t_n)])


def kernel(w1, b1, gamma, beta, w2, b2, x):
    del b1
    B, in_dim = x.shape
    hidden = w1.shape[1]
    out_dim = w2.shape[1]
    eps = 1e-5

    g2 = gamma.reshape(1, hidden)
    beta2 = beta.reshape(1, hidden)
    b2_2 = b2.reshape(1, out_dim)

    t_h = 512 if hidden % 512 == 0 else hidden
    n_h = hidden // t_h
    t_n = 512 if out_dim % 512 == 0 else out_dim
    n_n = out_dim // t_n
    steps = n_h + n_n
    nq = 4 if B % 32 == 0 else 1
    qb = B // nq

    def w1_idx(j):
        return (0, jnp.minimum(j, n_h - 1))

    def out_idx(j):
        return (0, jnp.clip(j - n_h, 0, n_n - 1))

    def xq_idx(i):
        return lambda j: (i, 0)

    body = functools.partial(_fused_mlp_kernel, eps=eps, inv_b=1.0 / B,
                             n_h=n_h, t_h=t_h, t_n=t_n, nq=nq)
    return pl.pallas_call(
        body,
        grid=(steps,),
        in_specs=[
            *[pl.BlockSpec((qb, in_dim), xq_idx(min(i, nq - 1)))
              for i in range(4)],
            pl.BlockSpec((in_dim, t_h), w1_idx),
            pl.BlockSpec((1, hidden), lambda j: (0, 0)),
            pl.BlockSpec((1, hidden), lambda j: (0, 0)),
            pl.BlockSpec(memory_space=pl.ANY),
            pl.BlockSpec((1, out_dim), lambda j: (0, 0)),
        ],
        out_specs=pl.BlockSpec((B, t_n), out_idx),
        out_shape=jax.ShapeDtypeStruct((B, out_dim), jnp.float32),
        scratch_shapes=[
            pltpu.VMEM((B, hidden), jnp.float32),
            pltpu.VMEM((hidden, out_dim), jnp.float32),
            pltpu.SemaphoreType.DMA,
        ],
        compiler_params=pltpu.CompilerParams(
            dimension_semantics=("arbitrary",)),
        cost_estimate=pl.CostEstimate(
            flops=2 * B * in_dim * hidden + 2 * B * hidden * out_dim,
            transcendentals=hidden,
            bytes_accessed=(B * in_dim + in_dim * hidden
                            + hidden * out_dim + B * out_dim) * 4,
        ),
    )(x, x, x, x, w1, g2, beta2, w2, b2_2)


# speedup vs baseline: 1.1142x; 1.0018x over previous
import functools

import jax
import jax.numpy as jnp
from jax import lax
from jax.experimental import pallas as pl
from jax.experimental.pallas import tpu as pltpu


def _fused_mlp_kernel(x0_ref, x1_ref, x2_ref, x3_ref, w1_ref, g_ref,
                      beta_ref, w2_hbm_ref, b2_ref, o_ref, hn_ref, w2_ref,
                      w2_sem, *, eps, inv_b, n_h, t_h, t_n, nq):
    j = pl.program_id(0)

    @pl.when(j == 0)
    def _start_w2_copy():
        pltpu.make_async_copy(w2_hbm_ref, w2_ref, w2_sem).start()

    @pl.when(j < n_h)
    def _hidden_tile():
        xqs = (x0_ref, x1_ref, x2_ref, x3_ref)[:nq]
        hs = [jnp.dot(xq[...], w1_ref[...],
                      preferred_element_type=jnp.float32) for xq in xqs]
        s1 = sum(jnp.sum(h, axis=0, keepdims=True) for h in hs)
        s2 = sum(jnp.sum(h * h, axis=0, keepdims=True) for h in hs)
        mean = s1 * inv_b
        var = s2 * inv_b - mean * mean
        col = pl.multiple_of(j * t_h, t_h)
        a = g_ref[:, pl.ds(col, t_h)] * lax.rsqrt(
            jnp.maximum(var, 0.0) + eps)
        c = beta_ref[:, pl.ds(col, t_h)] - mean * a
        q = x0_ref.shape[0]
        for i, h in enumerate(hs):
            hn_ref[i * q:(i + 1) * q, pl.ds(col, t_h)] = jnp.maximum(
                h * a + c, 0.0)

    @pl.when(j == n_h)
    def _finish_w2_copy():
        pltpu.make_async_copy(w2_hbm_ref, w2_ref, w2_sem).wait()

    @pl.when(j >= n_h)
    def _out_tile():
        col = pl.multiple_of((j - n_h) * t_n, t_n)
        o_ref[...] = (jnp.dot(hn_ref[...], w2_ref[:, pl.ds(col, t_n)],
                              preferred_element_type=jnp.float32)
                      + b2_ref[:, pl.ds(col, t_n)])


def kernel(w1, b1, gamma, beta, w2, b2, x):
    del b1
    B, in_dim = x.shape
    hidden = w1.shape[1]
    out_dim = w2.shape[1]
    eps = 1e-5

    g2 = gamma.reshape(1, hidden)
    beta2 = beta.reshape(1, hidden)
    b2_2 = b2.reshape(1, out_dim)

    t_h = 512 if hidden % 512 == 0 else hidden
    n_h = hidden // t_h
    t_n = 256 if out_dim % 256 == 0 else out_dim
    n_n = out_dim // t_n
    steps = n_h + n_n
    nq = 4 if B % 32 == 0 else 1
    qb = B // nq

    def w1_idx(j):
        return (0, jnp.minimum(j, n_h - 1))

    def out_idx(j):
        return (0, jnp.clip(j - n_h, 0, n_n - 1))

    def xq_idx(i):
        return lambda j: (i, 0)

    body = functools.partial(_fused_mlp_kernel, eps=eps, inv_b=1.0 / B,
                             n_h=n_h, t_h=t_h, t_n=t_n, nq=nq)
    return pl.pallas_call(
        body,
        grid=(steps,),
        in_specs=[
            *[pl.BlockSpec((qb, in_dim), xq_idx(min(i, nq - 1)))
              for i in range(4)],
            pl.BlockSpec((in_dim, t_h), w1_idx),
            pl.BlockSpec((1, hidden), lambda j: (0, 0)),
            pl.BlockSpec((1, hidden), lambda j: (0, 0)),
            pl.BlockSpec(memory_space=pl.ANY),
            pl.BlockSpec((1, out_dim), lambda j: (0, 0)),
        ],
        out_specs=pl.BlockSpec((B, t_n), out_idx),
        out_shape=jax.ShapeDtypeStruct((B, out_dim), jnp.float32),
        scratch_shapes=[
            pltpu.VMEM((B, hidden), jnp.float32),
            pltpu.VMEM((hidden, out_dim), jnp.float32),
            pltpu.SemaphoreType.DMA,
        ],
        compiler_params=pltpu.CompilerParams(
            dimension_semantics=("arbitrary",)),
        cost_estimate=pl.CostEstimate(
            flops=2 * B * in_dim * hidden + 2 * B * hidden * out_dim,
            transcendentals=hidden,
            bytes_accessed=(B * in_dim + in_dim * hidden
                            + hidden * out_dim + B * out_dim) * 4,
        ),
    )(x, x, x, x, w1, g2, beta2, w2, b2_2)


# speedup vs baseline: 1.1188x; 1.0041x over previous
import functools

import jax
import jax.numpy as jnp
from jax import lax
from jax.experimental import pallas as pl
from jax.experimental.pallas import tpu as pltpu


def _fused_mlp_kernel(x0_ref, x1_ref, x2_ref, x3_ref, w1_ref, g_ref,
                      beta_ref, w2_hbm_ref, b2_ref, o_ref, hn_ref, w2_ref,
                      w2_sem, *, eps, inv_b, n_h, t_h, t_n, nq):
    j = pl.program_id(0)

    @pl.when(j == 0)
    def _start_w2_copy():
        pltpu.make_async_copy(w2_hbm_ref, w2_ref, w2_sem).start()

    @pl.when(j < n_h)
    def _hidden_tile():
        xqs = (x0_ref, x1_ref, x2_ref, x3_ref)[:nq]
        hs = [jnp.dot(xq[...], w1_ref[...],
                      preferred_element_type=jnp.float32) for xq in xqs]
        s1 = sum(jnp.sum(h, axis=0, keepdims=True) for h in hs)
        s2 = sum(jnp.sum(h * h, axis=0, keepdims=True) for h in hs)
        mean = s1 * inv_b
        var = s2 * inv_b - mean * mean
        col = pl.multiple_of(j * t_h, t_h)
        a = g_ref[:, pl.ds(col, t_h)] * lax.rsqrt(
            jnp.maximum(var, 0.0) + eps)
        c = beta_ref[:, pl.ds(col, t_h)] - mean * a
        q = x0_ref.shape[0]
        for i, h in enumerate(hs):
            hn_ref[i * q:(i + 1) * q, pl.ds(col, t_h)] = jnp.maximum(
                h * a + c, 0.0)

    @pl.when(j == n_h)
    def _finish_w2_copy():
        pltpu.make_async_copy(w2_hbm_ref, w2_ref, w2_sem).wait()

    @pl.when(j >= n_h)
    def _out_tile():
        col = pl.multiple_of((j - n_h) * t_n, t_n)
        o_ref[...] = (jnp.dot(hn_ref[...], w2_ref[:, pl.ds(col, t_n)],
                              preferred_element_type=jnp.float32)
                      + b2_ref[:, pl.ds(col, t_n)])


def kernel(w1, b1, gamma, beta, w2, b2, x):
    del b1
    B, in_dim = x.shape
    hidden = w1.shape[1]
    out_dim = w2.shape[1]
    eps = 1e-5

    g2 = gamma.reshape(1, hidden)
    beta2 = beta.reshape(1, hidden)
    b2_2 = b2.reshape(1, out_dim)

    t_h = 512 if hidden % 512 == 0 else hidden
    n_h = hidden // t_h
    t_n = 512 if out_dim % 512 == 0 else out_dim
    n_n = out_dim // t_n
    steps = n_h + n_n
    nq = 4 if B % 32 == 0 else 1
    qb = B // nq

    def w1_idx(j):
        return (0, jnp.minimum(j, n_h - 1))

    def out_idx(j):
        return (0, jnp.clip(j - n_h, 0, n_n - 1))

    def xq_idx(i):
        return lambda j: (i, 0)

    body = functools.partial(_fused_mlp_kernel, eps=eps, inv_b=1.0 / B,
                             n_h=n_h, t_h=t_h, t_n=t_n, nq=nq)
    return pl.pallas_call(
        body,
        grid=(steps,),
        in_specs=[
            *[pl.BlockSpec((qb, in_dim), xq_idx(min(i, nq - 1)))
              for i in range(4)],
            pl.BlockSpec((in_dim, t_h), w1_idx),
            pl.BlockSpec((1, hidden), lambda j: (0, 0)),
            pl.BlockSpec((1, hidden), lambda j: (0, 0)),
            pl.BlockSpec(memory_space=pl.ANY),
            pl.BlockSpec((1, out_dim), lambda j: (0, 0)),
        ],
        out_specs=pl.BlockSpec((B, t_n), out_idx),
        out_shape=jax.ShapeDtypeStruct((B, out_dim), jnp.float32),
        scratch_shapes=[
            pltpu.VMEM((B, hidden), jnp.float32),
            pltpu.VMEM((hidden, out_dim), jnp.float32),
            pltpu.SemaphoreType.DMA,
        ],
        compiler_params=pltpu.CompilerParams(
            dimension_semantics=("arbitrary",)),
        cost_estimate=pl.CostEstimate(
            flops=2 * B * in_dim * hidden + 2 * B * hidden * out_dim,
            transcendentals=hidden,
            bytes_accessed=(B * in_dim + in_dim * hidden
                            + hidden * out_dim + B * out_dim) * 4,
        ),
    )(x, x, x, x, w1, g2, beta2, w2, b2_2)


# speedup vs baseline: 1.1191x; 1.0003x over previous
import functools

import jax
import jax.numpy as jnp
from jax import lax
from jax.experimental import pallas as pl
from jax.experimental.pallas import tpu as pltpu


def _fused_mlp_kernel(x0_ref, x1_ref, x2_ref, x3_ref, w1_ref, g_ref,
                      beta_ref, w2_hbm_ref, b2_ref, o_ref, hn_ref, w2_ref,
                      w2_sem, w2b_sem, *, eps, inv_b, n_h, t_h, t_n, nq):
    j = pl.program_id(0)

    h2 = w2_ref.shape[1] // 2

    @pl.when(j == 0)
    def _start_w2_copy():
        pltpu.make_async_copy(w2_hbm_ref.at[:, :h2], w2_ref.at[:, :h2],
                              w2_sem).start()
        pltpu.make_async_copy(w2_hbm_ref.at[:, h2:], w2_ref.at[:, h2:],
                              w2b_sem).start()

    @pl.when(j < n_h)
    def _hidden_tile():
        xqs = (x0_ref, x1_ref, x2_ref, x3_ref)[:nq]
        hs = [jnp.dot(xq[...], w1_ref[...],
                      preferred_element_type=jnp.float32) for xq in xqs]
        s1 = sum(jnp.sum(h, axis=0, keepdims=True) for h in hs)
        s2 = sum(jnp.sum(h * h, axis=0, keepdims=True) for h in hs)
        mean = s1 * inv_b
        var = s2 * inv_b - mean * mean
        col = pl.multiple_of(j * t_h, t_h)
        a = g_ref[:, pl.ds(col, t_h)] * lax.rsqrt(
            jnp.maximum(var, 0.0) + eps)
        c = beta_ref[:, pl.ds(col, t_h)] - mean * a
        q = x0_ref.shape[0]
        for i, h in enumerate(hs):
            hn_ref[i * q:(i + 1) * q, pl.ds(col, t_h)] = jnp.maximum(
                h * a + c, 0.0)

    @pl.when(j == n_h)
    def _finish_w2_copy():
        pltpu.make_async_copy(w2_hbm_ref.at[:, :h2], w2_ref.at[:, :h2],
                              w2_sem).wait()

    @pl.when(j == n_h + 1)
    def _finish_w2b_copy():
        pltpu.make_async_copy(w2_hbm_ref.at[:, h2:], w2_ref.at[:, h2:],
                              w2b_sem).wait()

    @pl.when(j >= n_h)
    def _out_tile():
        col = pl.multiple_of((j - n_h) * t_n, t_n)
        o_ref[...] = (jnp.dot(hn_ref[...], w2_ref[:, pl.ds(col, t_n)],
                              preferred_element_type=jnp.float32)
                      + b2_ref[:, pl.ds(col, t_n)])


def kernel(w1, b1, gamma, beta, w2, b2, x):
    del b1
    B, in_dim = x.shape
    hidden = w1.shape[1]
    out_dim = w2.shape[1]
    eps = 1e-5

    g2 = gamma.reshape(1, hidden)
    beta2 = beta.reshape(1, hidden)
    b2_2 = b2.reshape(1, out_dim)

    t_h = 512 if hidden % 512 == 0 else hidden
    n_h = hidden // t_h
    t_n = 512 if out_dim % 512 == 0 else out_dim
    n_n = out_dim // t_n
    steps = n_h + n_n
    nq = 4 if B % 32 == 0 else 1
    qb = B // nq

    def w1_idx(j):
        return (0, jnp.minimum(j, n_h - 1))

    def out_idx(j):
        return (0, jnp.clip(j - n_h, 0, n_n - 1))

    def xq_idx(i):
        return lambda j: (i, 0)

    body = functools.partial(_fused_mlp_kernel, eps=eps, inv_b=1.0 / B,
                             n_h=n_h, t_h=t_h, t_n=t_n, nq=nq)
    return pl.pallas_call(
        body,
        grid=(steps,),
        in_specs=[
            *[pl.BlockSpec((qb, in_dim), xq_idx(min(i, nq - 1)))
              for i in range(4)],
            pl.BlockSpec((in_dim, t_h), w1_idx),
            pl.BlockSpec((1, hidden), lambda j: (0, 0)),
            pl.BlockSpec((1, hidden), lambda j: (0, 0)),
            pl.BlockSpec(memory_space=pl.ANY),
            pl.BlockSpec((1, out_dim), lambda j: (0, 0)),
        ],
        out_specs=pl.BlockSpec((B, t_n), out_idx),
        out_shape=jax.ShapeDtypeStruct((B, out_dim), jnp.float32),
        scratch_shapes=[
            pltpu.VMEM((B, hidden), jnp.float32),
            pltpu.VMEM((hidden, out_dim), jnp.float32),
            pltpu.SemaphoreType.DMA,
            pltpu.SemaphoreType.DMA,
        ],
        compiler_params=pltpu.CompilerParams(
            dimension_semantics=("arbitrary",)),
        cost_estimate=pl.CostEstimate(
            flops=2 * B * in_dim * hidden + 2 * B * hidden * out_dim,
            transcendentals=hidden,
            bytes_accessed=(B * in_dim + in_dim * hidden
                            + hidden * out_dim + B * out_dim) * 4,
        ),
    )(x, x, x, x, w1, g2, beta2, w2, b2_2)
